# baseline (speedup 1.0000x reference)
import functools

import jax
import jax.numpy as jnp
from jax import lax
from jax.experimental import pallas as pl
from jax.experimental.pallas import tpu as pltpu
from jax.experimental.pallas import tpu_sc as plsc

BATCH = 16384
LIN_IN = 13
N_CATS = 26
EMB_DIM = 16
HIDDEN = 256
OUT = 6

PCATS = 32
WIDTH = PCATS * EMB_DIM
SLOTS = BATCH * PCATS
NC, NS = 2, 16
NW = NC * NS
SLOTS_W = SLOTS // NW
GROUP = 128
GROUPS_W = SLOTS_W // GROUP
GPC = 16
CHUNK = GPC * GROUP
CHUNKS_W = GROUPS_W // GPC


def _gather_body(emb_hbm, idx_hbm, out_hbm, idx_v, buf0, buf1, gsem, ws0, ws1):
    wid = lax.axis_index("s") * NC + lax.axis_index("c")
    base = wid * SLOTS_W
    pltpu.sync_copy(idx_hbm.at[pl.ds(wid * GROUPS_W, GROUPS_W)], idx_v)

    wb = [None] * CHUNKS_W
    for s in range(CHUNKS_W):
        buf, wsem = (buf0, ws0) if s % 2 == 0 else (buf1, ws1)
        if s >= 2:
            wb[s - 2].wait()
        copies = []
        for j in range(GPC):
            copies.append(pltpu.async_copy(
                emb_hbm.at[idx_v.at[s * GPC + j]],
                buf.at[pl.ds(j * GROUP, GROUP)],
                gsem,
            ))
        for c in copies:
            c.wait()
        wb[s] = pltpu.async_copy(
            buf, out_hbm.at[pl.ds(base + s * CHUNK, CHUNK)], wsem)
    wb[CHUNKS_W - 2].wait()
    wb[CHUNKS_W - 1].wait()


_gather = functools.partial(
    pl.kernel,
    mesh=plsc.VectorSubcoreMesh(core_axis_name="c", subcore_axis_name="s"),
    compiler_params=pltpu.CompilerParams(use_tc_tiling_on_sc=False),
    out_type=jax.ShapeDtypeStruct((SLOTS, EMB_DIM), jnp.float32),
    scratch_types=[
        pltpu.VMEM((GROUPS_W, GROUP), jnp.int32),
        pltpu.VMEM((CHUNK, EMB_DIM), jnp.float32),
        pltpu.VMEM((CHUNK, EMB_DIM), jnp.float32),
        pltpu.SemaphoreType.DMA,
        pltpu.SemaphoreType.DMA,
        pltpu.SemaphoreType.DMA,
    ],
)(_gather_body)


BB = 512
RB = BB // 8


def _mlp_body(x1_ref, e4_ref, w1a_ref, w1bp_ref, b1_ref, w2_ref, b2_ref, out_ref):
    acc = jnp.dot(x1_ref[...], w1a_ref[...], preferred_element_type=jnp.float32)
    for t in range(WIDTH // 128):
        et = e4_ref[:, t].reshape(BB, 128)
        acc += jnp.dot(et, w1bp_ref[t], preferred_element_type=jnp.float32)
    h = jnp.tanh(acc + b1_ref[...])
    out_ref[...] = (
        jnp.dot(h, w2_ref[...], preferred_element_type=jnp.float32) + b2_ref[...]
    )


def _mlp(x1, e4, w1a, w1bp, b1, w2, b2):
    return pl.pallas_call(
        _mlp_body,
        grid=(BATCH // BB,),
        in_specs=[
            pl.BlockSpec((BB, LIN_IN), lambda i: (i, 0)),
            pl.BlockSpec((RB, WIDTH // 128, 8, 128), lambda i: (i, 0, 0, 0)),
            pl.BlockSpec((LIN_IN, HIDDEN), lambda i: (0, 0)),
            pl.BlockSpec((WIDTH // 128, 128, HIDDEN), lambda i: (0, 0, 0)),
            pl.BlockSpec((1, HIDDEN), lambda i: (0, 0)),
            pl.BlockSpec((HIDDEN, OUT), lambda i: (0, 0)),
            pl.BlockSpec((1, OUT), lambda i: (0, 0)),
        ],
        out_specs=pl.BlockSpec((BB, OUT), lambda i: (i, 0)),
        out_shape=jax.ShapeDtypeStruct((BATCH, OUT), jnp.float32),
    )(x1, e4, w1a, w1bp, b1, w2, b2)


def kernel(x1, x2, emb, W1, b1, W2, b2):
    x2i = x2.astype(jnp.int32)
    x2p = jnp.pad(x2i, ((0, 0), (0, PCATS - N_CATS)))
    idx = (
        x2p.reshape(BATCH // 8, 8, PCATS // 8, 8)
        .transpose(0, 2, 1, 3)
        .reshape(SLOTS // GROUP, GROUP)
    )
    e = _gather(emb, idx)
    e4 = e.reshape(BATCH // 8, WIDTH // 128, 8, 128)

    w1bp = jnp.concatenate(
        [W1[LIN_IN:], jnp.zeros((WIDTH - (N_CATS * EMB_DIM), HIDDEN), jnp.float32)]
    ).reshape(WIDTH // 128, 128, HIDDEN)
    return _mlp(
        x1,
        e4,
        W1[:LIN_IN],
        w1bp,
        b1.reshape(1, HIDDEN),
        W2,
        b2.reshape(1, OUT),
    )


# speedup vs baseline: 1.0002x; 1.0002x over previous
import functools

import jax
import jax.numpy as jnp
from jax import lax
from jax.experimental import pallas as pl
from jax.experimental.pallas import tpu as pltpu
from jax.experimental.pallas import tpu_sc as plsc

BATCH = 16384
LIN_IN = 13
N_CATS = 26
EMB_DIM = 16
HIDDEN = 256
OUT = 6

PCATS = 32
WIDTH = PCATS * EMB_DIM
SLOTS = BATCH * PCATS
NC, NS = 2, 16
NW = NC * NS
SLOTS_W = SLOTS // NW
GROUP = 128
GROUPS_W = SLOTS_W // GROUP
GPC = 16
CHUNK = GPC * GROUP
CHUNKS_W = GROUPS_W // GPC


def _gather_body(emb_hbm, idx_hbm, out_hbm, idx_v, buf, gsem):
    wid = lax.axis_index("s") * NC + lax.axis_index("c")
    base = wid * SLOTS_W
    pltpu.sync_copy(idx_hbm.at[pl.ds(wid * GROUPS_W, GROUPS_W)], idx_v)

    def chunk_body(s, carry):
        copies = []
        for j in range(GPC):
            copies.append(pltpu.async_copy(
                emb_hbm.at[idx_v.at[s * GPC + j]],
                buf.at[pl.ds(j * GROUP, GROUP)],
                gsem,
            ))
        for c in copies:
            c.wait()
        pltpu.sync_copy(buf, out_hbm.at[pl.ds(base + s * CHUNK, CHUNK)])
        return carry

    lax.fori_loop(0, CHUNKS_W, chunk_body, 0)


_gather = functools.partial(
    pl.kernel,
    mesh=plsc.VectorSubcoreMesh(core_axis_name="c", subcore_axis_name="s"),
    compiler_params=pltpu.CompilerParams(use_tc_tiling_on_sc=False),
    out_type=jax.ShapeDtypeStruct((SLOTS, EMB_DIM), jnp.float32),
    scratch_types=[
        pltpu.VMEM((GROUPS_W, GROUP), jnp.int32),
        pltpu.VMEM((CHUNK, EMB_DIM), jnp.float32),
        pltpu.SemaphoreType.DMA,
    ],
)(_gather_body)


BB = 512
RB = BB // 8


def _mlp_body(x1_ref, e4_ref, w1a_ref, w1bp_ref, b1_ref, w2_ref, b2_ref, out_ref):
    acc = jnp.dot(x1_ref[...], w1a_ref[...], preferred_element_type=jnp.float32)
    for t in range(WIDTH // 128):
        et = e4_ref[:, t].reshape(BB, 128)
        acc += jnp.dot(et, w1bp_ref[t], preferred_element_type=jnp.float32)
    h = jnp.tanh(acc + b1_ref[...])
    out_ref[...] = (
        jnp.dot(h, w2_ref[...], preferred_element_type=jnp.float32) + b2_ref[...]
    )


def _mlp(x1, e4, w1a, w1bp, b1, w2, b2):
    return pl.pallas_call(
        _mlp_body,
        grid=(BATCH // BB,),
        in_specs=[
            pl.BlockSpec((BB, LIN_IN), lambda i: (i, 0)),
            pl.BlockSpec((RB, WIDTH // 128, 8, 128), lambda i: (i, 0, 0, 0)),
            pl.BlockSpec((LIN_IN, HIDDEN), lambda i: (0, 0)),
            pl.BlockSpec((WIDTH // 128, 128, HIDDEN), lambda i: (0, 0, 0)),
            pl.BlockSpec((1, HIDDEN), lambda i: (0, 0)),
            pl.BlockSpec((HIDDEN, OUT), lambda i: (0, 0)),
            pl.BlockSpec((1, OUT), lambda i: (0, 0)),
        ],
        out_specs=pl.BlockSpec((BB, OUT), lambda i: (i, 0)),
        out_shape=jax.ShapeDtypeStruct((BATCH, OUT), jnp.float32),
    )(x1, e4, w1a, w1bp, b1, w2, b2)


def kernel(x1, x2, emb, W1, b1, W2, b2):
    x2i = x2.astype(jnp.int32)
    x2p = jnp.pad(x2i, ((0, 0), (0, PCATS - N_CATS)))
    idx = (
        x2p.reshape(BATCH // 8, 8, PCATS // 8, 8)
        .transpose(0, 2, 1, 3)
        .reshape(SLOTS // GROUP, GROUP)
    )
    e = _gather(emb, idx)
    e4 = e.reshape(BATCH // 8, WIDTH // 128, 8, 128)

    w1bp = jnp.concatenate(
        [W1[LIN_IN:], jnp.zeros((WIDTH - (N_CATS * EMB_DIM), HIDDEN), jnp.float32)]
    ).reshape(WIDTH // 128, 128, HIDDEN)
    return _mlp(
        x1,
        e4,
        W1[:LIN_IN],
        w1bp,
        b1.reshape(1, HIDDEN),
        W2,
        b2.reshape(1, OUT),
    )


# speedup vs baseline: 1.8304x; 1.8301x over previous
import functools

import jax
import jax.numpy as jnp
from jax import lax
from jax.experimental import pallas as pl
from jax.experimental.pallas import tpu as pltpu
from jax.experimental.pallas import tpu_sc as plsc

BATCH = 16384
LIN_IN = 13
N_CATS = 26
EMB_DIM = 16
HIDDEN = 256
OUT = 6

PCATS = 32
WIDTH = PCATS * EMB_DIM
SLOTS = BATCH * PCATS
NC, NS = 2, 16
NW = NC * NS
SLOTS_W = SLOTS // NW
GROUP = 128
GROUPS_W = SLOTS_W // GROUP
GPC = 16
CHUNK = GPC * GROUP
CHUNKS_W = GROUPS_W // GPC


def _gather_body(emb_hbm, idx_hbm, out_hbm, idx_v, buf, gsem):
    wid = lax.axis_index("s") * NC + lax.axis_index("c")
    base = wid * SLOTS_W
    pltpu.sync_copy(idx_hbm.at[pl.ds(wid * GROUPS_W, GROUPS_W)], idx_v)

    def chunk_body(s, carry):
        copies = []
        for j in range(GPC):
            copies.append(pltpu.async_copy(
                emb_hbm.at[idx_v.at[s * GPC + j]],
                buf.at[pl.ds(j * GROUP, GROUP)],
                gsem,
            ))
        for c in copies:
            c.wait()
        pltpu.sync_copy(buf, out_hbm.at[pl.ds(base + s * CHUNK, CHUNK)])
        return carry

    lax.fori_loop(0, CHUNKS_W, chunk_body, 0)


_gather = functools.partial(
    pl.kernel,
    mesh=plsc.VectorSubcoreMesh(core_axis_name="c", subcore_axis_name="s"),
    compiler_params=pltpu.CompilerParams(use_tc_tiling_on_sc=False),
    out_type=jax.ShapeDtypeStruct((SLOTS, EMB_DIM), jnp.float32),
    scratch_types=[
        pltpu.VMEM((GROUPS_W, GROUP), jnp.int32),
        pltpu.VMEM((CHUNK, EMB_DIM), jnp.float32),
        pltpu.SemaphoreType.DMA,
    ],
)(_gather_body)


BB = 512
RB = BB // 8


def _mlp_body(x1_ref, e4_ref, w1a_ref, w1bp_ref, b1_ref, w2_ref, b2_ref, out_ref):
    acc = jnp.dot(x1_ref[...], w1a_ref[...], preferred_element_type=jnp.float32)
    for t in range(WIDTH // 128):
        et = e4_ref[:, t].reshape(BB, 128)
        acc += jnp.dot(et, w1bp_ref[t], preferred_element_type=jnp.float32)
    h = jnp.tanh(acc + b1_ref[...])
    out_ref[...] = (
        jnp.dot(h, w2_ref[...], preferred_element_type=jnp.float32) + b2_ref[...]
    )


def _mlp(x1, e4, w1a, w1bp, b1, w2, b2):
    return pl.pallas_call(
        _mlp_body,
        grid=(BATCH // BB,),
        in_specs=[
            pl.BlockSpec((BB, LIN_IN), lambda i: (i, 0)),
            pl.BlockSpec((RB, WIDTH // 128, 8, 128), lambda i: (i, 0, 0, 0)),
            pl.BlockSpec((LIN_IN, HIDDEN), lambda i: (0, 0)),
            pl.BlockSpec((WIDTH // 128, 128, HIDDEN), lambda i: (0, 0, 0)),
            pl.BlockSpec((1, HIDDEN), lambda i: (0, 0)),
            pl.BlockSpec((HIDDEN, OUT), lambda i: (0, 0)),
            pl.BlockSpec((1, OUT), lambda i: (0, 0)),
        ],
        out_specs=pl.BlockSpec((BB, OUT), lambda i: (i, 0)),
        out_shape=jax.ShapeDtypeStruct((BATCH, OUT), jnp.float32),
    )(x1, e4, w1a, w1bp, b1, w2, b2)


def kernel(x1, x2, emb, W1, b1, W2, b2):
    x2i = x2.astype(jnp.int32)
    x2p = jnp.concatenate([x2i, x2i[:, : PCATS - N_CATS]], axis=1)
    idx = (
        x2p.reshape(BATCH // 8, 8, PCATS // 8, 8)
        .transpose(0, 2, 1, 3)
        .reshape(SLOTS // GROUP, GROUP)
    )
    e = _gather(emb, idx)
    e4 = e.reshape(BATCH // 8, WIDTH // 128, 8, 128)

    w1bp = jnp.concatenate(
        [W1[LIN_IN:], jnp.zeros((WIDTH - (N_CATS * EMB_DIM), HIDDEN), jnp.float32)]
    ).reshape(WIDTH // 128, 128, HIDDEN)
    return _mlp(
        x1,
        e4,
        W1[:LIN_IN],
        w1bp,
        b1.reshape(1, HIDDEN),
        W2,
        b2.reshape(1, OUT),
    )


# speedup vs baseline: 2.0619x; 1.1265x over previous
import functools

import jax
import jax.numpy as jnp
from jax import lax
from jax.experimental import pallas as pl
from jax.experimental.pallas import tpu as pltpu
from jax.experimental.pallas import tpu_sc as plsc

BATCH = 16384
LIN_IN = 13
N_CATS = 26
EMB_DIM = 16
HIDDEN = 256
OUT = 6

PCATS = 32
WIDTH = PCATS * EMB_DIM
SLOTS = BATCH * PCATS
NC, NS = 2, 16
NW = NC * NS
SLOTS_W = SLOTS // NW
GROUP = 128
GROUPS_W = SLOTS_W // GROUP
GPC = 16
CHUNK = GPC * GROUP
CHUNKS_W = GROUPS_W // GPC


def _gather_body(emb_hbm, idx_hbm, out_hbm, idx_v, buf, gsem):
    wid = lax.axis_index("s") * NC + lax.axis_index("c")
    base = wid * SLOTS_W
    pltpu.sync_copy(idx_hbm.at[pl.ds(wid * GROUPS_W, GROUPS_W)], idx_v)

    def chunk_body(s, carry):
        copies = []
        for j in range(GPC):
            copies.append(pltpu.async_copy(
                emb_hbm.at[idx_v.at[s * GPC + j]],
                buf.at[pl.ds(j * GROUP, GROUP)],
                gsem,
            ))
        for c in copies:
            c.wait()
        pltpu.sync_copy(buf, out_hbm.at[pl.ds(base + s * CHUNK, CHUNK)])
        return carry

    lax.fori_loop(0, CHUNKS_W, chunk_body, 0)


_gather = functools.partial(
    pl.kernel,
    mesh=plsc.VectorSubcoreMesh(core_axis_name="c", subcore_axis_name="s"),
    compiler_params=pltpu.CompilerParams(use_tc_tiling_on_sc=False),
    out_type=jax.ShapeDtypeStruct((SLOTS, EMB_DIM), jnp.float32),
    scratch_types=[
        pltpu.VMEM((GROUPS_W, GROUP), jnp.int32),
        pltpu.VMEM((CHUNK, EMB_DIM), jnp.float32),
        pltpu.SemaphoreType.DMA,
    ],
)(_gather_body)


VOCAB = 1000000
TCB = 8192
TGRID = -(-VOCAB // TCB)


def _transpose_body(embt_ref, out_ref):
    t3 = embt_ref[...].T.reshape(TCB // 8, 8, EMB_DIM)
    out_ref[...] = jnp.concatenate([t3[:, m, :] for m in range(8)], axis=1)


def _transpose(embt):
    return pl.pallas_call(
        _transpose_body,
        grid=(TGRID,),
        in_specs=[pl.BlockSpec((EMB_DIM, TCB), lambda i: (0, i))],
        out_specs=pl.BlockSpec((TCB // 8, 128), lambda i: (i, 0)),
        out_shape=jax.ShapeDtypeStruct((VOCAB // 8, 128), jnp.float32),
    )(embt)


BB = 512
RB = BB // 8


def _mlp_body(x1_ref, e4_ref, w1a_ref, w1bp_ref, b1_ref, w2_ref, b2_ref, out_ref):
    acc = jnp.dot(x1_ref[...], w1a_ref[...], preferred_element_type=jnp.float32)
    for t in range(WIDTH // 128):
        et = e4_ref[:, t].reshape(BB, 128)
        acc += jnp.dot(et, w1bp_ref[t], preferred_element_type=jnp.float32)
    h = jnp.tanh(acc + b1_ref[...])
    out_ref[...] = (
        jnp.dot(h, w2_ref[...], preferred_element_type=jnp.float32) + b2_ref[...]
    )


def _mlp(x1, e4, w1a, w1bp, b1, w2, b2):
    return pl.pallas_call(
        _mlp_body,
        grid=(BATCH // BB,),
        in_specs=[
            pl.BlockSpec((BB, LIN_IN), lambda i: (i, 0)),
            pl.BlockSpec((RB, WIDTH // 128, 8, 128), lambda i: (i, 0, 0, 0)),
            pl.BlockSpec((LIN_IN, HIDDEN), lambda i: (0, 0)),
            pl.BlockSpec((WIDTH // 128, 128, HIDDEN), lambda i: (0, 0, 0)),
            pl.BlockSpec((1, HIDDEN), lambda i: (0, 0)),
            pl.BlockSpec((HIDDEN, OUT), lambda i: (0, 0)),
            pl.BlockSpec((1, OUT), lambda i: (0, 0)),
        ],
        out_specs=pl.BlockSpec((BB, OUT), lambda i: (i, 0)),
        out_shape=jax.ShapeDtypeStruct((BATCH, OUT), jnp.float32),
    )(x1, e4, w1a, w1bp, b1, w2, b2)


def kernel(x1, x2, emb, W1, b1, W2, b2):
    x2i = x2.astype(jnp.int32)
    x2p = jnp.concatenate([x2i, x2i[:, : PCATS - N_CATS]], axis=1)
    idx = (
        x2p.reshape(BATCH // 8, 8, PCATS // 8, 8)
        .transpose(0, 2, 1, 3)
        .reshape(SLOTS // GROUP, GROUP)
    )
    emb_lin = _transpose(emb.T).reshape(VOCAB, EMB_DIM)
    e = _gather(emb_lin, idx)
    e4 = e.reshape(BATCH // 8, WIDTH // 128, 8, 128)

    w1bp = jnp.concatenate(
        [W1[LIN_IN:], jnp.zeros((WIDTH - (N_CATS * EMB_DIM), HIDDEN), jnp.float32)]
    ).reshape(WIDTH // 128, 128, HIDDEN)
    return _mlp(
        x1,
        e4,
        W1[:LIN_IN],
        w1bp,
        b1.reshape(1, HIDDEN),
        W2,
        b2.reshape(1, OUT),
    )


# speedup vs baseline: 4.2802x; 2.0759x over previous
import functools

import jax
import jax.numpy as jnp
from jax import lax
from jax.experimental import pallas as pl
from jax.experimental.pallas import tpu as pltpu
from jax.experimental.pallas import tpu_sc as plsc

BATCH = 16384
LIN_IN = 13
N_CATS = 26
EMB_DIM = 16
HIDDEN = 256
OUT = 6
VOCAB = 1000000

PCATS = 32
WIDTH = PCATS * EMB_DIM
SLOTS = BATCH * PCATS
NC, NS = 2, 16
NW = NC * NS
SLOTS_W = SLOTS // NW
GROUP = 128
GROUPS_W = SLOTS_W // GROUP
TROWS_W = GROUPS_W // 2
GPC = 16
CHUNK = GPC * GROUP
CHUNKS_W = GROUPS_W // GPC

TCB = 8192
TGRID = -(-VOCAB // TCB)
VPAD = -(-VOCAB // 1024) * 1024


def _repack_body(embt_ref, out_ref):
    for mb in range(TCB // 1024):
        xm = embt_ref[:, mb * 1024:(mb + 1) * 1024].reshape(EMB_DIM, 8, 128)
        v = jnp.swapaxes(xm, 0, 1).reshape(128, 128)
        out_ref[mb * 128:(mb + 1) * 128, :] = v.T


def _repack(embt):
    return pl.pallas_call(
        _repack_body,
        grid=(TGRID,),
        in_specs=[pl.BlockSpec((EMB_DIM, TCB), lambda i: (0, i))],
        out_specs=pl.BlockSpec((TCB // 8, 128), lambda i: (i, 0)),
        out_shape=jax.ShapeDtypeStruct((VPAD // 8, 128), jnp.float32),
    )(embt)


IDXBB = 2048
IDXRB = IDXBB // 8


def _idx_body(x2t_ref, out_ref):
    x = x2t_ref[...]
    q = jnp.concatenate([x, x[: PCATS - N_CATS]], axis=0)
    t = (
        q.reshape(2, 2, 8, IDXRB, 8)
        .transpose(0, 3, 1, 4, 2)
        .reshape(2, IDXRB, GROUP)
    )
    out_ref[...] = (t & -1024) | ((t & 127) << 3) | ((t >> 7) & 7)


def _idx(x2t):
    return pl.pallas_call(
        _idx_body,
        grid=(BATCH // IDXBB,),
        in_specs=[pl.BlockSpec((N_CATS, IDXBB), lambda i: (0, i))],
        out_specs=pl.BlockSpec((2, IDXRB, GROUP), lambda i: (0, i, 0)),
        out_shape=jax.ShapeDtypeStruct((2, SLOTS // GROUP // 2, GROUP), jnp.int32),
    )(x2t)


def _gather_body(emb_hbm, idx_hbm, out_hbm, idx_v, buf, gsem):
    wid = lax.axis_index("s") * NC + lax.axis_index("c")
    base = wid * SLOTS_W
    pltpu.sync_copy(idx_hbm.at[pl.ds(wid * TROWS_W, TROWS_W)],
                    idx_v.at[pl.ds(0, TROWS_W)])
    pltpu.sync_copy(idx_hbm.at[pl.ds(SLOTS // GROUP // 2 + wid * TROWS_W, TROWS_W)],
                    idx_v.at[pl.ds(TROWS_W, TROWS_W)])

    def chunk_body(s, carry):
        copies = []
        for rr in range(GPC // 2):
            for h in range(2):
                copies.append(pltpu.async_copy(
                    emb_hbm.at[idx_v.at[h * TROWS_W + s * (GPC // 2) + rr]],
                    buf.at[pl.ds((rr * 2 + h) * GROUP, GROUP)],
                    gsem,
                ))
        for c in copies:
            c.wait()
        pltpu.sync_copy(buf, out_hbm.at[pl.ds(base + s * CHUNK, CHUNK)])
        return carry

    lax.fori_loop(0, CHUNKS_W, chunk_body, 0)


_gather = functools.partial(
    pl.kernel,
    mesh=plsc.VectorSubcoreMesh(core_axis_name="c", subcore_axis_name="s"),
    compiler_params=pltpu.CompilerParams(use_tc_tiling_on_sc=False),
    out_type=jax.ShapeDtypeStruct((SLOTS, EMB_DIM), jnp.float32),
    scratch_types=[
        pltpu.VMEM((GROUPS_W, GROUP), jnp.int32),
        pltpu.VMEM((CHUNK, EMB_DIM), jnp.float32),
        pltpu.SemaphoreType.DMA,
    ],
)(_gather_body)


BB = 512
RB = BB // 8


def _mlp_body(x1_ref, e4_ref, w1a_ref, w1bp_ref, b1_ref, w2_ref, b2_ref, out_ref):
    acc = jnp.dot(x1_ref[...], w1a_ref[...], preferred_element_type=jnp.float32)
    for t in range(WIDTH // 128):
        et = e4_ref[:, t].reshape(BB, 128)
        acc += jnp.dot(et, w1bp_ref[t], preferred_element_type=jnp.float32)
    h = jnp.tanh(acc + b1_ref[...])
    out_ref[...] = (
        jnp.dot(h, w2_ref[...], preferred_element_type=jnp.float32) + b2_ref[...]
    )


def _mlp(x1, e4, w1a, w1bp, b1, w2, b2):
    return pl.pallas_call(
        _mlp_body,
        grid=(BATCH // BB,),
        in_specs=[
            pl.BlockSpec((BB, LIN_IN), lambda i: (i, 0)),
            pl.BlockSpec((RB, WIDTH // 128, 8, 128), lambda i: (i, 0, 0, 0)),
            pl.BlockSpec((LIN_IN, HIDDEN), lambda i: (0, 0)),
            pl.BlockSpec((WIDTH // 128, 128, HIDDEN), lambda i: (0, 0, 0)),
            pl.BlockSpec((1, HIDDEN), lambda i: (0, 0)),
            pl.BlockSpec((HIDDEN, OUT), lambda i: (0, 0)),
            pl.BlockSpec((1, OUT), lambda i: (0, 0)),
        ],
        out_specs=pl.BlockSpec((BB, OUT), lambda i: (i, 0)),
        out_shape=jax.ShapeDtypeStruct((BATCH, OUT), jnp.float32),
    )(x1, e4, w1a, w1bp, b1, w2, b2)


def kernel(x1, x2, emb, W1, b1, W2, b2):
    emb_lin = _repack(emb.T).reshape(VPAD, EMB_DIM)
    idx = _idx(x2.astype(jnp.int32).T).reshape(SLOTS // GROUP, GROUP)
    e = _gather(emb_lin, idx)
    e4 = e.reshape(BATCH // 8, WIDTH // 128, 8, 128)

    w1bp = jnp.concatenate(
        [W1[LIN_IN:], jnp.zeros((WIDTH - (N_CATS * EMB_DIM), HIDDEN), jnp.float32)]
    ).reshape(WIDTH // 128, 128, HIDDEN)
    return _mlp(
        x1,
        e4,
        W1[:LIN_IN],
        w1bp,
        b1.reshape(1, HIDDEN),
        W2,
        b2.reshape(1, OUT),
    )


# speedup vs baseline: 4.4580x; 1.0415x over previous
import functools

import jax
import jax.numpy as jnp
from jax import lax
from jax.experimental import pallas as pl
from jax.experimental.pallas import tpu as pltpu
from jax.experimental.pallas import tpu_sc as plsc

BATCH = 16384
LIN_IN = 13
N_CATS = 26
EMB_DIM = 16
HIDDEN = 256
OUT = 6
VOCAB = 1000000

PCATS = 32
WIDTH = PCATS * EMB_DIM
SLOTS = BATCH * PCATS
NC, NS = 2, 16
NW = NC * NS
SLOTS_W = SLOTS // NW
GROUP = 128
GROUPS_W = SLOTS_W // GROUP
TROWS_W = GROUPS_W // 2
GPC = 16
CHUNK = GPC * GROUP
CHUNKS_W = GROUPS_W // GPC

TCB = 8192
TGRID = -(-VOCAB // TCB)
VPAD = -(-VOCAB // 1024) * 1024


def _repack_body(embt_ref, out_ref):
    for mb in range(TCB // 1024):
        xm = embt_ref[:, mb * 1024:(mb + 1) * 1024].reshape(EMB_DIM, 8, 128)
        v = jnp.swapaxes(xm, 0, 1).reshape(128, 128)
        out_ref[mb * 128:(mb + 1) * 128, :] = v.T


def _repack(embt):
    return pl.pallas_call(
        _repack_body,
        grid=(TGRID,),
        in_specs=[pl.BlockSpec((EMB_DIM, TCB), lambda i: (0, i))],
        out_specs=pl.BlockSpec((TCB // 8, 128), lambda i: (i, 0)),
        out_shape=jax.ShapeDtypeStruct((VPAD // 8, 128), jnp.float32),
    )(embt)


IDXBB = 2048
IDXRB = IDXBB // 8


def _idx_body(x2t_ref, out_ref):
    x = x2t_ref[...]
    q = jnp.concatenate([x, x[: PCATS - N_CATS]], axis=0)
    t = (
        q.reshape(2, 2, 8, IDXRB, 8)
        .transpose(0, 3, 1, 4, 2)
        .reshape(2, IDXRB, GROUP)
    )
    out_ref[...] = (t & -1024) | ((t & 127) << 3) | ((t >> 7) & 7)


def _idx(x2t):
    return pl.pallas_call(
        _idx_body,
        grid=(BATCH // IDXBB,),
        in_specs=[pl.BlockSpec((N_CATS, IDXBB), lambda i: (0, i))],
        out_specs=pl.BlockSpec((2, IDXRB, GROUP), lambda i: (0, i, 0)),
        out_shape=jax.ShapeDtypeStruct((2, SLOTS // GROUP // 2, GROUP), jnp.int32),
    )(x2t)


def _gather_body(emb_hbm, idx_hbm, out_hbm, idx_v, buf, gsem):
    wid = lax.axis_index("s") * NC + lax.axis_index("c")
    base = wid * SLOTS_W
    pltpu.sync_copy(idx_hbm.at[pl.ds(wid * TROWS_W, TROWS_W)],
                    idx_v.at[pl.ds(0, TROWS_W)])
    pltpu.sync_copy(idx_hbm.at[pl.ds(SLOTS // GROUP // 2 + wid * TROWS_W, TROWS_W)],
                    idx_v.at[pl.ds(TROWS_W, TROWS_W)])

    def chunk_body(s, carry):
        copies = []
        for rr in range(GPC // 2):
            for h in range(2):
                copies.append(pltpu.async_copy(
                    emb_hbm.at[idx_v.at[h * TROWS_W + s * (GPC // 2) + rr]],
                    buf.at[pl.ds((rr * 2 + h) * GROUP, GROUP)],
                    gsem,
                ))
        for c in copies:
            c.wait()
        pltpu.sync_copy(buf, out_hbm.at[pl.ds(base + s * CHUNK, CHUNK)])
        return carry

    lax.fori_loop(0, CHUNKS_W, chunk_body, 0)


_gather = functools.partial(
    pl.kernel,
    mesh=plsc.VectorSubcoreMesh(core_axis_name="c", subcore_axis_name="s"),
    compiler_params=pltpu.CompilerParams(use_tc_tiling_on_sc=False),
    out_type=jax.ShapeDtypeStruct((SLOTS, EMB_DIM), jnp.float32),
    scratch_types=[
        pltpu.VMEM((GROUPS_W, GROUP), jnp.int32),
        pltpu.VMEM((CHUNK, EMB_DIM), jnp.float32),
        pltpu.SemaphoreType.DMA,
    ],
)(_gather_body)


BB = 512
RB = BB // 8


def _mlp_body(x1t_ref, e4_ref, w1at_ref, w1bp_ref, b1_ref, w2_ref, b2_ref, out_ref):
    acc = jnp.dot(x1t_ref[...].T, w1at_ref[...], preferred_element_type=jnp.float32)
    for t in range(WIDTH // 128):
        et = e4_ref[:, t].reshape(BB, 128)
        acc += jnp.dot(et, w1bp_ref[t], preferred_element_type=jnp.float32)
    h = jnp.tanh(acc + b1_ref[...])
    out_ref[...] = (
        jnp.dot(h, w2_ref[...], preferred_element_type=jnp.float32) + b2_ref[...]
    ).T


def _mlp(x1t, e4, w1at, w1bp, b1, w2, b2):
    return pl.pallas_call(
        _mlp_body,
        grid=(BATCH // BB,),
        in_specs=[
            pl.BlockSpec((LIN_IN, BB), lambda i: (0, i)),
            pl.BlockSpec((RB, WIDTH // 128, 8, 128), lambda i: (i, 0, 0, 0)),
            pl.BlockSpec((LIN_IN, HIDDEN), lambda i: (0, 0)),
            pl.BlockSpec((WIDTH // 128, 128, HIDDEN), lambda i: (0, 0, 0)),
            pl.BlockSpec((1, HIDDEN), lambda i: (0, 0)),
            pl.BlockSpec((HIDDEN, OUT), lambda i: (0, 0)),
            pl.BlockSpec((1, OUT), lambda i: (0, 0)),
        ],
        out_specs=pl.BlockSpec((OUT, BB), lambda i: (0, i)),
        out_shape=jax.ShapeDtypeStruct((OUT, BATCH), jnp.float32),
    )(x1t, e4, w1at, w1bp, b1, w2, b2)


def kernel(x1, x2, emb, W1, b1, W2, b2):
    emb_lin = _repack(emb.T).reshape(VPAD, EMB_DIM)
    idx = _idx(x2.astype(jnp.int32).T).reshape(SLOTS // GROUP, GROUP)
    e = _gather(emb_lin, idx)
    e4 = e.reshape(BATCH // 8, WIDTH // 128, 8, 128)

    w1bp = jnp.concatenate(
        [W1[LIN_IN:], jnp.zeros((WIDTH - (N_CATS * EMB_DIM), HIDDEN), jnp.float32)]
    ).reshape(WIDTH // 128, 128, HIDDEN)
    return _mlp(
        x1.T,
        e4,
        W1[:LIN_IN],
        w1bp,
        b1.reshape(1, HIDDEN),
        W2,
        b2.reshape(1, OUT),
    ).T
